# Initial kernel scaffold; baseline (speedup 1.0000x reference)
import jax
import jax.numpy as jnp
from jax.experimental import pallas as pl


def kernel(x0, x1):
    raise NotImplementedError("write your pallas kernel here")


# speedup vs baseline: 13.8285x; 13.8285x over previous
import functools

import jax
import jax.numpy as jnp
from jax import lax
from jax.experimental import pallas as pl
from jax.experimental.pallas import tpu as pltpu
from jax.experimental.pallas import tpu_sc as plsc

B, N, S, K = 8, 8192, 512, 32
L = 16
NC, NS = 2, 16
NW = NC * NS
NQ = B * S
QPT = NQ // NW
TPB = S // QPT
NCH = N // L
QB = 4
NG = QPT // QB
SEG = (2, 4, 8, 16, 32, 64, 128, 192, 256, 320, 384, 448, 512)
CB = 64 * L + L
EPS = 1e-5

_mesh = plsc.VectorSubcoreMesh(
    core_axis_name="c", subcore_axis_name="s", num_cores=NC, num_subcores=NS)


def _rev(x):
    return lax.rev(x, (0,))


def _twosum(a, b):
    s = a + b
    bp = s - a
    ap = s - bp
    return s, (a - ap) + (b - bp)


def _bf16_round(x):
    b = plsc.bitcast(x, jnp.int32)
    r = (b + jnp.int32(0x7FFF) + ((b >> 16) & 1)) & jnp.int32(-0x10000)
    return plsc.bitcast(r, jnp.float32)


def _sort2(d, i):
    s_i, s_d = plsc.sort_key_val(i, d)
    return plsc.sort_key_val(s_d, s_i)


def _bcast_last(v):
    return v.at[jnp.full((L,), L - 1, jnp.int32)].get(mode="promise_in_bounds")


def kernel_factory():
    def body(x0t_hbm, qb_hbm, out_hbm, idx_hbm,
             x0_v, qb_v, x0r_v, bb_v, ib_v, out_v, idx_v):
        wid = lax.axis_index("s") * NC + lax.axis_index("c")
        b = wid // TPB
        s0 = (wid % TPB) * QPT
        pltpu.sync_copy(x0t_hbm.at[b], x0_v)
        pltpu.sync_copy(qb_hbm.at[wid], qb_v)

        def round_points(c, _):
            px = x0_v[0, pl.ds(c * L, L)]
            py = x0_v[1, pl.ds(c * L, L)]
            pz = x0_v[2, pl.ds(c * L, L)]
            x0r_v[0, pl.ds(c * L, L)] = _bf16_round(px)
            x0r_v[1, pl.ds(c * L, L)] = _bf16_round(py)
            x0r_v[2, pl.ds(c * L, L)] = _bf16_round(pz)
            bb_v[0, pl.ds(c * L, L)] = px * px + py * py + pz * pz
            return 0

        lax.fori_loop(0, NCH, round_points, 0)

        inf = jnp.full((L,), jnp.inf, jnp.float32)
        zero = jnp.zeros((L,), jnp.int32)
        lane = lax.iota(jnp.int32, L)

        def merge16(Ci, valid, A, Ai, Bv, Bi, q):
            qxr, qyr, qzr, aa = q
            sel0 = jnp.full((L,), 0, jnp.int32)
            gx = plsc.load_gather(x0r_v, [sel0, Ci])
            gy = plsc.load_gather(x0r_v, [jnp.full((L,), 1, jnp.int32), Ci])
            gz = plsc.load_gather(x0r_v, [jnp.full((L,), 2, jnp.int32), Ci])
            gb = plsc.load_gather(bb_v, [sel0, Ci])
            p0, p1, p2 = qxr * gx, qyr * gy, qzr * gz
            s1, e1 = _twosum(p1, p2)
            s2, e2 = _twosum(p0, s1)
            abx = s2 + (e1 + e2)
            dx = (aa + gb) - 2.0 * abx
            if valid is not None:
                dx = jnp.where(valid, dx, jnp.inf)
            cs, cis = plsc.sort_key_val(dx, Ci)
            cr, cir = _rev(cs), _rev(cis)
            m = Bv <= cr
            lo = jnp.where(m, Bv, cr)
            loi = jnp.where(m, Bi, cir)
            Dv, Di = _sort2(lo, loi)
            dr, dir_ = _rev(Dv), _rev(Di)
            m2 = (A < dr) | ((A == dr) & (Ai < dir_))
            lo2 = jnp.where(m2, A, dr)
            loi2 = jnp.where(m2, Ai, dir_)
            hi2 = jnp.where(m2, dr, A)
            hii2 = jnp.where(m2, dir_, Ai)
            A2, Ai2 = _sort2(lo2, loi2)
            B2, Bi2 = _sort2(hi2, hii2)
            return A2, Ai2, B2, Bi2

        def do_qgroup(qg, _):
            qr = []
            for j in range(QB):
                qi0 = qg * QB + j
                qx = qb_v[0, pl.ds(qi0 * L, L)]
                qy = qb_v[1, pl.ds(qi0 * L, L)]
                qz = qb_v[2, pl.ds(qi0 * L, L)]
                aa = qx * qx + qy * qy + qz * qz
                qr.append((_bf16_round(qx), _bf16_round(qy), _bf16_round(qz),
                           aa))

            def chunk(c, car):
                cnts, tvs = car
                pxr = x0r_v[0, pl.ds(c * L, L)]
                pyr = x0r_v[1, pl.ds(c * L, L)]
                pzr = x0r_v[2, pl.ds(c * L, L)]
                bb = bb_v[0, pl.ds(c * L, L)]
                cib = lane + c * L
                ncnts = []
                for j in range(QB):
                    qxr, qyr, qzr, aa = qr[j]
                    ab = qxr * pxr + (qyr * pyr + qzr * pzr)
                    dj = (aa + bb) - 2.0 * ab
                    mj = dj < tvs[j]
                    pos = (cnts[j] + plsc.cumsum(mj.astype(jnp.int32))) - 1
                    plsc.store_scatter(
                        ib_v, [jnp.full((L,), j, jnp.int32), pos], cib,
                        mask=mj)
                    pc = plsc.all_reduce_population_count(mj)
                    ncnts.append(cnts[j] + pc)
                return (tuple(ncnts), tvs)

            st = [(inf, zero, inf, zero) for _ in range(QB)]
            carry = (tuple(zero for _ in range(QB)),
                     tuple(inf for _ in range(QB)))
            prev = 0
            for gi, seg_end in enumerate(SEG):
                carry = lax.fori_loop(prev, seg_end, chunk, carry)
                prev = seg_end
                final = gi == len(SEG) - 1
                cnts, tvs = carry
                ncnts, ntvs = [], []
                for j in range(QB):
                    n = cnts[j][0]
                    A, Ai, Bv, Bi = st[j]

                    def wcond(ws):
                        return (ws[0] + L) <= n

                    def wbody(ws, j=j):
                        g, A, Ai, Bv, Bi = ws
                        Ci = ib_v[j, pl.ds(g, L)]
                        A, Ai, Bv, Bi = merge16(Ci, None, A, Ai, Bv, Bi,
                                                qr[j])
                        return (g + L, A, Ai, Bv, Bi)

                    g, A, Ai, Bv, Bi = lax.while_loop(
                        wcond, wbody, (jnp.int32(0), A, Ai, Bv, Bi))
                    if final:
                        r = n - g
                        Ci = ib_v[j, pl.ds(g, L)]
                        lanem = lane < jnp.broadcast_to(r, (L,))
                        A, Ai, Bv, Bi = merge16(Ci, lanem, A, Ai, Bv, Bi,
                                                qr[j])
                        ncnts.append(zero)
                    else:
                        ib_v[j, pl.ds(0, L)] = ib_v[j, pl.ds(g, L)]
                        ncnts.append(jnp.broadcast_to(n - g, (L,)))
                    st[j] = (A, Ai, Bv, Bi)
                    ntvs.append(_bcast_last(Bv) + jnp.float32(EPS))
                carry = (tuple(ncnts), tuple(ntvs))

            for j in range(QB):
                qi0 = qg * QB + j
                qx = qb_v[0, pl.ds(qi0 * L, L)]
                qy = qb_v[1, pl.ds(qi0 * L, L)]
                qz = qb_v[2, pl.ds(qi0 * L, L)]
                A, Ai, Bv, Bi = st[j]
                for half, I in ((0, Ai), (1, Bi)):
                    o = qi0 * K + half * L
                    idx_v[0, pl.ds(o, L)] = I
                    for c3, q in ((0, qx), (1, qy), (2, qz)):
                        sel = jnp.full((L,), c3, jnp.int32)
                        g2 = plsc.load_gather(x0_v, [sel, I])
                        out_v[c3, pl.ds(o, L)] = g2 - q
                        out_v[3 + c3, pl.ds(o, L)] = q
            return 0

        lax.fori_loop(0, NG, do_qgroup, 0)
        pltpu.sync_copy(out_v, out_hbm.at[b, :, pl.ds(s0 * K, QPT * K)])
        pltpu.sync_copy(idx_v.at[0], idx_hbm.at[b, pl.ds(s0 * K, QPT * K)])

    return functools.partial(
        pl.kernel,
        out_type=(
            jax.ShapeDtypeStruct((B, 6, S * K), jnp.float32),
            jax.ShapeDtypeStruct((B, S * K), jnp.int32),
        ),
        mesh=_mesh,
        scratch_types=[
            pltpu.VMEM((3, N), jnp.float32),
            pltpu.VMEM((3, QPT * L), jnp.float32),
            pltpu.VMEM((3, N), jnp.float32),
            pltpu.VMEM((1, N), jnp.float32),
            pltpu.VMEM((QB, CB), jnp.int32),
            pltpu.VMEM((6, QPT * K), jnp.float32),
            pltpu.VMEM((1, QPT * K), jnp.int32),
        ],
        compiler_params=pltpu.CompilerParams(needs_layout_passes=False),
    )(body)


_sc_call = kernel_factory()


def kernel(x0, x1):
    x0t = jnp.transpose(x0, (0, 2, 1))
    xq = x1.reshape(NQ, 3)
    qb = (jnp.broadcast_to(xq[:, :, None], (NQ, 3, L))
          .reshape(NW, QPT, 3, L)
          .transpose(0, 2, 1, 3)
          .reshape(NW, 3, QPT * L))
    out, idxs = _sc_call(x0t, qb)
    centroids = jnp.transpose(x1, (0, 2, 1))
    return out.reshape(B, 6, S, K), centroids, idxs.reshape(B, S, K)
